# baseline (speedup 1.0000x reference)
import jax
import jax.numpy as jnp
from jax.experimental import pallas as pl
from jax.experimental.pallas import tpu as pltpu

_BM = 128


def _gtconv_body(h_ref, s_ref, x_ref, o_ref):
    acc = h_ref[0, 0] * jnp.dot(
        s_ref[0].astype(jnp.bfloat16), x_ref[...],
        preferred_element_type=jnp.float32)
    for i in range(1, s_ref.shape[0]):
        acc = acc + h_ref[0, i] * jnp.dot(
            s_ref[i].astype(jnp.bfloat16), x_ref[...],
            preferred_element_type=jnp.float32)
    o_ref[...] = acc


@jax.jit
def kernel(x, S_powers, h):
    K, N, _ = S_powers.shape
    D = x.shape[1]
    grid = (N // _BM,)
    return pl.pallas_call(
        _gtconv_body,
        grid=grid,
        in_specs=[
            pl.BlockSpec((1, K), lambda i: (0, 0)),
            pl.BlockSpec((K, _BM, N), lambda i: (0, i, 0)),
            pl.BlockSpec((N, D), lambda i: (0, 0)),
        ],
        out_specs=pl.BlockSpec((_BM, D), lambda i: (i, 0)),
        out_shape=jax.ShapeDtypeStruct((N, D), jnp.float32),
        compiler_params=pltpu.CompilerParams(
            dimension_semantics=("arbitrary",),
        ),
    )(h.reshape(1, K), S_powers, x.astype(jnp.bfloat16))


# speedup vs baseline: 1.0620x; 1.0620x over previous
import jax
import jax.numpy as jnp
from jax.experimental import pallas as pl
from jax.experimental.pallas import tpu as pltpu

_BM = 128


def _gtconv_body(h_ref, s_ref, x_ref, o_ref):
    hb = h_ref[0, 0] * s_ref[0]
    for i in range(1, s_ref.shape[0]):
        hb = hb + h_ref[0, i] * s_ref[i]
    o_ref[...] = jnp.dot(hb, x_ref[...],
                         preferred_element_type=jnp.float32)


@jax.jit
def kernel(x, S_powers, h):
    K, N, _ = S_powers.shape
    D = x.shape[1]
    grid = (N // _BM,)
    return pl.pallas_call(
        _gtconv_body,
        grid=grid,
        in_specs=[
            pl.BlockSpec((1, K), lambda i: (0, 0)),
            pl.BlockSpec((K, _BM, N), lambda i: (0, i, 0)),
            pl.BlockSpec((N, D), lambda i: (0, 0)),
        ],
        out_specs=pl.BlockSpec((_BM, D), lambda i: (i, 0)),
        out_shape=jax.ShapeDtypeStruct((N, D), jnp.float32),
        compiler_params=pltpu.CompilerParams(
            dimension_semantics=("arbitrary",),
        ),
    )(h.reshape(1, K), S_powers, x)


# speedup vs baseline: 1.0648x; 1.0026x over previous
import jax
import jax.numpy as jnp
from jax.experimental import pallas as pl
from jax.experimental.pallas import tpu as pltpu

_BM = 128


def _gtconv_body(h_ref, s_ref, x_ref, o_ref):
    hb = h_ref[0, 0] * s_ref[0]
    for i in range(1, s_ref.shape[0]):
        hb = hb + h_ref[0, i] * s_ref[i]
    o_ref[...] = jnp.dot(hb, x_ref[...],
                         preferred_element_type=jnp.float32)


@jax.jit
def kernel(x, S_powers, h):
    K, N, _ = S_powers.shape
    D = x.shape[1]
    grid = (N // _BM,)
    return pl.pallas_call(
        _gtconv_body,
        grid=grid,
        in_specs=[
            pl.BlockSpec((1, K), lambda i: (0, 0)),
            pl.BlockSpec((K, _BM, N), lambda i: (0, i, 0)),
            pl.BlockSpec((N, D), lambda i: (0, 0)),
        ],
        out_specs=pl.BlockSpec((_BM, D), lambda i: (i, 0)),
        out_shape=jax.ShapeDtypeStruct((N, D), jnp.float32),
        compiler_params=pltpu.CompilerParams(
            dimension_semantics=("parallel",),
        ),
    )(h.reshape(1, K), S_powers, x)
